# Initial kernel scaffold; baseline (speedup 1.0000x reference)
import jax
import jax.numpy as jnp
from jax.experimental import pallas as pl


def kernel(block_outputs, pos_edge_index, neg_edge_index):
    raise NotImplementedError("write your pallas kernel here")


# speedup vs baseline: 3.3019x; 3.3019x over previous
import functools

import jax
import jax.numpy as jnp
from jax import lax
from jax.experimental import pallas as pl
from jax.experimental.pallas import tpu as pltpu
from jax.experimental.pallas import tpu_sc as plsc

N_NODES = 10000
D = 128
E_POS = 320000
E_NEG = 320000
E_TOT = E_POS + E_NEG

NC = 2
NS = 16
L = 16
NW = NC * NS

E_PER_W = E_TOT // NW
CHUNK = 80
NCHUNK = E_PER_W // CHUNK

_sc_mesh = plsc.VectorSubcoreMesh(core_axis_name="c", subcore_axis_name="s")


@functools.partial(
    pl.kernel,
    out_type=jax.ShapeDtypeStruct((E_TOT, L), jnp.float32),
    mesh=_sc_mesh,
    scratch_types=[
        pltpu.VMEM((E_PER_W,), jnp.int32),
        pltpu.VMEM((E_PER_W,), jnp.int32),
        pltpu.VMEM((CHUNK, D), jnp.float32),
        pltpu.VMEM((CHUNK, D), jnp.float32),
        pltpu.VMEM((CHUNK, L), jnp.float32),
        pltpu.SemaphoreType.DMA,
        pltpu.SemaphoreType.DMA,
    ],
)
def _edge_dot_sc(table_hbm, src_hbm, dst_hbm, out_hbm,
                 src_v, dst_v, srow, drow, part, sem_s, sem_d):
    wid = lax.axis_index("s") * NC + lax.axis_index("c")
    base = wid * E_PER_W
    pltpu.sync_copy(src_hbm.at[pl.ds(base, E_PER_W)], src_v)
    pltpu.sync_copy(dst_hbm.at[pl.ds(base, E_PER_W)], dst_v)

    def chunk_body(g, carry):
        off = g * CHUNK
        cp_s = pltpu.async_copy(
            table_hbm.at[src_v.at[pl.ds(off, CHUNK)]], srow, sem_s)
        cp_d = pltpu.async_copy(
            table_hbm.at[dst_v.at[pl.ds(off, CHUNK)]], drow, sem_d)
        cp_s.wait()
        cp_d.wait()

        def edge_body(e, c):
            acc = srow[e, pl.ds(0, L)] * drow[e, pl.ds(0, L)]
            for k in range(1, D // L):
                acc = acc + srow[e, pl.ds(k * L, L)] * drow[e, pl.ds(k * L, L)]
            part[e, :] = acc
            return c

        lax.fori_loop(0, CHUNK, edge_body, 0)
        pltpu.sync_copy(part, out_hbm.at[pl.ds(base + off, CHUNK)])
        return carry

    lax.fori_loop(0, NCHUNK, chunk_body, 0)


_BLK = 8000
_NBLK = E_TOT // _BLK


def _loss_body(p_ref, out_ref):
    pid = pl.program_id(0)
    x = p_ref[...]
    s = jnp.sum(x, axis=1, keepdims=True)
    rows = pid * _BLK + lax.broadcasted_iota(jnp.int32, (_BLK, 1), 0)
    label = (rows < E_POS).astype(jnp.float32)
    loss = jnp.maximum(s, 0.0) - s * label + jnp.log1p(jnp.exp(-jnp.abs(s)))
    bsum = jnp.sum(loss)

    @pl.when(pid == 0)
    def _init():
        out_ref[0, 0] = 0.0

    out_ref[0, 0] += bsum

    @pl.when(pid == pl.num_programs(0) - 1)
    def _fini():
        out_ref[0, 0] = out_ref[0, 0] / E_TOT


_loss_tc = pl.pallas_call(
    _loss_body,
    grid=(_NBLK,),
    in_specs=[pl.BlockSpec((_BLK, L), lambda i: (i, 0))],
    out_specs=pl.BlockSpec(
        (1, 1), lambda i: (0, 0), memory_space=pltpu.SMEM),
    out_shape=jax.ShapeDtypeStruct((1, 1), jnp.float32),
)


def kernel(block_outputs, pos_edge_index, neg_edge_index):
    src = jnp.concatenate(
        [pos_edge_index[0], neg_edge_index[0]]).astype(jnp.int32)
    dst = jnp.concatenate(
        [pos_edge_index[1], neg_edge_index[1]]).astype(jnp.int32)
    partials = _edge_dot_sc(block_outputs, src, dst)
    loss = _loss_tc(partials)
    return loss[0, 0]


# speedup vs baseline: 4.7540x; 1.4398x over previous
import functools

import jax
import jax.numpy as jnp
from jax import lax
from jax.experimental import pallas as pl
from jax.experimental.pallas import tpu as pltpu
from jax.experimental.pallas import tpu_sc as plsc

N_NODES = 10000
D = 128
E_POS = 320000
E_NEG = 320000
E_TOT = E_POS + E_NEG

NC = 2
NS = 16
L = 16
NW = NC * NS

E_PER_W = E_TOT // NW
CHUNK = 80
NCHUNK = E_PER_W // CHUNK

_sc_mesh = plsc.VectorSubcoreMesh(core_axis_name="c", subcore_axis_name="s")


@functools.partial(
    pl.kernel,
    out_type=jax.ShapeDtypeStruct((E_TOT, L), jnp.float32),
    mesh=_sc_mesh,
    scratch_types=[
        pltpu.VMEM((E_PER_W,), jnp.int32),
        pltpu.VMEM((E_PER_W,), jnp.int32),
        pltpu.VMEM((CHUNK, D), jnp.float32),
        pltpu.VMEM((CHUNK, D), jnp.float32),
        pltpu.VMEM((CHUNK, D), jnp.float32),
        pltpu.VMEM((CHUNK, D), jnp.float32),
        pltpu.VMEM((CHUNK, L), jnp.float32),
        pltpu.VMEM((CHUNK, L), jnp.float32),
        pltpu.SemaphoreType.DMA,
        pltpu.SemaphoreType.DMA,
        pltpu.SemaphoreType.DMA,
        pltpu.SemaphoreType.DMA,
        pltpu.SemaphoreType.DMA,
        pltpu.SemaphoreType.DMA,
    ],
)
def _edge_dot_sc(table_hbm, src_hbm, dst_hbm, out_hbm,
                 src_v, dst_v, srow_a, drow_a, srow_b, drow_b,
                 part_a, part_b, sem_sa, sem_da, sem_sb, sem_db,
                 sem_oa, sem_ob):
    wid = lax.axis_index("s") * NC + lax.axis_index("c")
    base = wid * E_PER_W
    pltpu.sync_copy(src_hbm.at[pl.ds(base, E_PER_W)], src_v)
    pltpu.sync_copy(dst_hbm.at[pl.ds(base, E_PER_W)], dst_v)

    def issue(g, srow, drow, sem_s, sem_d):
        off = g * CHUNK
        pltpu.async_copy(table_hbm.at[src_v.at[pl.ds(off, CHUNK)]],
                         srow, sem_s)
        pltpu.async_copy(table_hbm.at[dst_v.at[pl.ds(off, CHUNK)]],
                         drow, sem_d)

    def wait_rows(srow, drow, sem_s, sem_d):
        pltpu.make_async_copy(
            table_hbm.at[src_v.at[pl.ds(0, CHUNK)]], srow, sem_s).wait()
        pltpu.make_async_copy(
            table_hbm.at[dst_v.at[pl.ds(0, CHUNK)]], drow, sem_d).wait()

    def compute(srow, drow, part):
        @plsc.parallel_loop(0, CHUNK, unroll=4)
        def _edge(e):
            acc = srow[e, pl.ds(0, L)] * drow[e, pl.ds(0, L)]
            for k in range(1, D // L):
                acc = acc + srow[e, pl.ds(k * L, L)] * drow[e, pl.ds(k * L, L)]
            part[e, :] = acc

    def wait_out(part, sem_o):
        pltpu.make_async_copy(
            part, out_hbm.at[pl.ds(base, CHUNK)], sem_o).wait()

    issue(0, srow_a, drow_a, sem_sa, sem_da)

    def pair_body(i, carry):
        ga = 2 * i
        gb = ga + 1
        issue(gb, srow_b, drow_b, sem_sb, sem_db)
        wait_rows(srow_a, drow_a, sem_sa, sem_da)

        @pl.when(i > 0)
        def _drain_a():
            wait_out(part_a, sem_oa)

        compute(srow_a, drow_a, part_a)
        pltpu.async_copy(part_a, out_hbm.at[pl.ds(base + ga * CHUNK, CHUNK)],
                         sem_oa)

        @pl.when(gb + 1 < NCHUNK)
        def _next_a():
            issue(gb + 1, srow_a, drow_a, sem_sa, sem_da)

        wait_rows(srow_b, drow_b, sem_sb, sem_db)

        @pl.when(i > 0)
        def _drain_b():
            wait_out(part_b, sem_ob)

        compute(srow_b, drow_b, part_b)
        pltpu.async_copy(part_b, out_hbm.at[pl.ds(base + gb * CHUNK, CHUNK)],
                         sem_ob)
        return carry

    lax.fori_loop(0, NCHUNK // 2, pair_body, 0)
    wait_out(part_a, sem_oa)
    wait_out(part_b, sem_ob)


_BLK = 8000
_NBLK = E_TOT // _BLK


def _loss_body(p_ref, out_ref):
    pid = pl.program_id(0)
    x = p_ref[...]
    s = jnp.sum(x, axis=1, keepdims=True)
    rows = pid * _BLK + lax.broadcasted_iota(jnp.int32, (_BLK, 1), 0)
    label = (rows < E_POS).astype(jnp.float32)
    loss = jnp.maximum(s, 0.0) - s * label + jnp.log1p(jnp.exp(-jnp.abs(s)))
    bsum = jnp.sum(loss)

    @pl.when(pid == 0)
    def _init():
        out_ref[0, 0] = 0.0

    out_ref[0, 0] += bsum

    @pl.when(pid == pl.num_programs(0) - 1)
    def _fini():
        out_ref[0, 0] = out_ref[0, 0] / E_TOT


_loss_tc = pl.pallas_call(
    _loss_body,
    grid=(_NBLK,),
    in_specs=[pl.BlockSpec((_BLK, L), lambda i: (i, 0))],
    out_specs=pl.BlockSpec(
        (1, 1), lambda i: (0, 0), memory_space=pltpu.SMEM),
    out_shape=jax.ShapeDtypeStruct((1, 1), jnp.float32),
)


def kernel(block_outputs, pos_edge_index, neg_edge_index):
    src = jnp.concatenate(
        [pos_edge_index[0], neg_edge_index[0]]).astype(jnp.int32)
    dst = jnp.concatenate(
        [pos_edge_index[1], neg_edge_index[1]]).astype(jnp.int32)
    partials = _edge_dot_sc(block_outputs, src, dst)
    loss = _loss_tc(partials)
    return loss[0, 0]


# speedup vs baseline: 5.2329x; 1.1008x over previous
import functools

import jax
import jax.numpy as jnp
from jax import lax
from jax.experimental import pallas as pl
from jax.experimental.pallas import tpu as pltpu
from jax.experimental.pallas import tpu_sc as plsc

N_NODES = 10000
D = 128
E_POS = 320000
E_NEG = 320000
E_TOT = E_POS + E_NEG
E_HALF = E_TOT // 2

NC = 2
NS = 16
L = 16

E_PER_W = E_HALF // NS
CHUNK = 80
NCHUNK = E_PER_W // CHUNK

_sc_mesh = plsc.VectorSubcoreMesh(core_axis_name="c", subcore_axis_name="s")


@functools.partial(
    pl.kernel,
    out_type=(
        jax.ShapeDtypeStruct((E_HALF, L), jnp.float32),
        jax.ShapeDtypeStruct((E_HALF, L), jnp.float32),
    ),
    mesh=_sc_mesh,
    scratch_types=[
        pltpu.VMEM((E_PER_W,), jnp.int32),
        pltpu.VMEM((E_PER_W,), jnp.int32),
        pltpu.VMEM((CHUNK, D), jnp.float32),
        pltpu.VMEM((CHUNK, D), jnp.float32),
        pltpu.VMEM((CHUNK, D), jnp.float32),
        pltpu.VMEM((CHUNK, D), jnp.float32),
        pltpu.VMEM((CHUNK, L), jnp.float32),
        pltpu.VMEM((CHUNK, L), jnp.float32),
        pltpu.SemaphoreType.DMA,
        pltpu.SemaphoreType.DMA,
        pltpu.SemaphoreType.DMA,
        pltpu.SemaphoreType.DMA,
        pltpu.SemaphoreType.DMA,
        pltpu.SemaphoreType.DMA,
    ],
)
def _edge_dot_sc(table_hbm, src_hbm, dst_hbm, out0_hbm, out1_hbm,
                 src_v, dst_v, srow_a, drow_a, srow_b, drow_b,
                 part_a, part_b, sem_sa, sem_da, sem_sb, sem_db,
                 sem_oa, sem_ob):
    cid = lax.axis_index("c")
    sid = lax.axis_index("s")
    hbase = sid * E_PER_W
    gbase = cid * E_HALF + hbase
    pltpu.sync_copy(src_hbm.at[pl.ds(gbase, E_PER_W)], src_v)
    pltpu.sync_copy(dst_hbm.at[pl.ds(gbase, E_PER_W)], dst_v)

    def issue(g, srow, drow, sem_s, sem_d):
        off = g * CHUNK
        pltpu.async_copy(table_hbm.at[src_v.at[pl.ds(off, CHUNK)]],
                         srow, sem_s)
        pltpu.async_copy(table_hbm.at[dst_v.at[pl.ds(off, CHUNK)]],
                         drow, sem_d)

    def wait_rows(srow, drow, sem_s, sem_d):
        pltpu.make_async_copy(
            table_hbm.at[src_v.at[pl.ds(0, CHUNK)]], srow, sem_s).wait()
        pltpu.make_async_copy(
            table_hbm.at[dst_v.at[pl.ds(0, CHUNK)]], drow, sem_d).wait()

    def compute(srow, drow, part):
        @plsc.parallel_loop(0, CHUNK, unroll=4)
        def _edge(e):
            acc = srow[e, pl.ds(0, L)] * drow[e, pl.ds(0, L)]
            for k in range(1, D // L):
                acc = acc + srow[e, pl.ds(k * L, L)] * drow[e, pl.ds(k * L, L)]
            part[e, :] = acc

    def run(out_hbm):
        def wait_out(part, sem_o):
            pltpu.make_async_copy(
                part, out_hbm.at[pl.ds(hbase, CHUNK)], sem_o).wait()

        issue(0, srow_a, drow_a, sem_sa, sem_da)

        def pair_body(i, carry):
            ga = 2 * i
            gb = ga + 1
            issue(gb, srow_b, drow_b, sem_sb, sem_db)
            wait_rows(srow_a, drow_a, sem_sa, sem_da)

            @pl.when(i > 0)
            def _drain_a():
                wait_out(part_a, sem_oa)

            compute(srow_a, drow_a, part_a)
            pltpu.async_copy(
                part_a, out_hbm.at[pl.ds(hbase + ga * CHUNK, CHUNK)], sem_oa)

            @pl.when(gb + 1 < NCHUNK)
            def _next_a():
                issue(gb + 1, srow_a, drow_a, sem_sa, sem_da)

            wait_rows(srow_b, drow_b, sem_sb, sem_db)

            @pl.when(i > 0)
            def _drain_b():
                wait_out(part_b, sem_ob)

            compute(srow_b, drow_b, part_b)
            pltpu.async_copy(
                part_b, out_hbm.at[pl.ds(hbase + gb * CHUNK, CHUNK)], sem_ob)
            return carry

        lax.fori_loop(0, NCHUNK // 2, pair_body, 0)
        wait_out(part_a, sem_oa)
        wait_out(part_b, sem_ob)

    @pl.when(cid == 0)
    def _core0():
        run(out0_hbm)

    @pl.when(cid == 1)
    def _core1():
        run(out1_hbm)


_BLK = 8000
_NBLK = E_HALF // _BLK


def _loss_body(p_pos_ref, p_neg_ref, out_ref):
    pid = pl.program_id(0)
    s0 = jnp.sum(p_pos_ref[...], axis=1, keepdims=True)
    s1 = jnp.sum(p_neg_ref[...], axis=1, keepdims=True)
    l0 = jnp.maximum(-s0, 0.0) + jnp.log1p(jnp.exp(-jnp.abs(s0)))
    l1 = jnp.maximum(s1, 0.0) + jnp.log1p(jnp.exp(-jnp.abs(s1)))
    bsum = jnp.sum(l0) + jnp.sum(l1)

    @pl.when(pid == 0)
    def _init():
        out_ref[0, 0] = 0.0

    out_ref[0, 0] += bsum

    @pl.when(pid == pl.num_programs(0) - 1)
    def _fini():
        out_ref[0, 0] = out_ref[0, 0] / E_TOT


_loss_tc = pl.pallas_call(
    _loss_body,
    grid=(_NBLK,),
    in_specs=[
        pl.BlockSpec((_BLK, L), lambda i: (i, 0)),
        pl.BlockSpec((_BLK, L), lambda i: (i, 0)),
    ],
    out_specs=pl.BlockSpec(
        (1, 1), lambda i: (0, 0), memory_space=pltpu.SMEM),
    out_shape=jax.ShapeDtypeStruct((1, 1), jnp.float32),
)


def kernel(block_outputs, pos_edge_index, neg_edge_index):
    src = jnp.concatenate(
        [pos_edge_index[0], neg_edge_index[0]]).astype(jnp.int32)
    dst = jnp.concatenate(
        [pos_edge_index[1], neg_edge_index[1]]).astype(jnp.int32)
    p_pos, p_neg = _edge_dot_sc(block_outputs, src, dst)
    loss = _loss_tc(p_pos, p_neg)
    return loss[0, 0]


# speedup vs baseline: 5.3270x; 1.0180x over previous
import functools

import jax
import jax.numpy as jnp
from jax import lax
from jax.experimental import pallas as pl
from jax.experimental.pallas import tpu as pltpu
from jax.experimental.pallas import tpu_sc as plsc

N_NODES = 10000
D = 128
E_POS = 320000
E_NEG = 320000
E_TOT = E_POS + E_NEG
E_HALF = E_TOT // 2

NC = 2
NS = 16
L = 16

E_PER_W = E_HALF // NS
CHUNK = 40
NCHUNK = E_PER_W // CHUNK

_sc_mesh = plsc.VectorSubcoreMesh(core_axis_name="c", subcore_axis_name="s")


@functools.partial(
    pl.kernel,
    out_type=(
        jax.ShapeDtypeStruct((E_HALF, L), jnp.float32),
        jax.ShapeDtypeStruct((E_HALF, L), jnp.float32),
    ),
    mesh=_sc_mesh,
    scratch_types=[
        pltpu.VMEM_SHARED((N_NODES, D), jnp.float32),
        pltpu.VMEM((CHUNK,), jnp.int32),
        pltpu.VMEM((CHUNK,), jnp.int32),
        pltpu.VMEM((CHUNK,), jnp.int32),
        pltpu.VMEM((CHUNK,), jnp.int32),
        pltpu.VMEM((CHUNK, D), jnp.float32),
        pltpu.VMEM((CHUNK, D), jnp.float32),
        pltpu.VMEM((CHUNK, D), jnp.float32),
        pltpu.VMEM((CHUNK, D), jnp.float32),
        pltpu.VMEM((CHUNK, L), jnp.float32),
        pltpu.VMEM((CHUNK, L), jnp.float32),
        pltpu.SemaphoreType.DMA,
        pltpu.SemaphoreType.DMA,
        pltpu.SemaphoreType.DMA,
        pltpu.SemaphoreType.DMA,
        pltpu.SemaphoreType.DMA,
        pltpu.SemaphoreType.DMA,
        pltpu.SemaphoreType.DMA,
        pltpu.SemaphoreType.DMA,
    ],
)
def _edge_dot_sc(table_hbm, src_hbm, dst_hbm, out0_hbm, out1_hbm,
                 table_sh, sidx_a, didx_a, sidx_b, didx_b,
                 srow_a, drow_a, srow_b, drow_b,
                 part_a, part_b,
                 sem_ia, sem_ib, sem_sa, sem_da, sem_sb, sem_db,
                 sem_oa, sem_ob):
    cid = lax.axis_index("c")
    sid = lax.axis_index("s")
    hbase = sid * E_PER_W
    gbase = cid * E_HALF + hbase

    @pl.when(sid == 0)
    def _stage_table():
        pltpu.sync_copy(table_hbm, table_sh)

    plsc.subcore_barrier()

    def issue_idx(g, sidx, didx, sem_i):
        off = gbase + g * CHUNK
        pltpu.async_copy(src_hbm.at[pl.ds(off, CHUNK)], sidx, sem_i)
        pltpu.async_copy(dst_hbm.at[pl.ds(off, CHUNK)], didx, sem_i)

    def wait_idx(sidx, didx, sem_i):
        pltpu.make_async_copy(
            src_hbm.at[pl.ds(gbase, CHUNK)], sidx, sem_i).wait()
        pltpu.make_async_copy(
            dst_hbm.at[pl.ds(gbase, CHUNK)], didx, sem_i).wait()

    def issue_rows(sidx, didx, srow, drow, sem_s, sem_d):
        pltpu.async_copy(table_sh.at[sidx], srow, sem_s)
        pltpu.async_copy(table_sh.at[didx], drow, sem_d)

    def wait_rows(sidx, didx, srow, drow, sem_s, sem_d):
        pltpu.make_async_copy(table_sh.at[sidx], srow, sem_s).wait()
        pltpu.make_async_copy(table_sh.at[didx], drow, sem_d).wait()

    def compute(srow, drow, part):
        @plsc.parallel_loop(0, CHUNK, unroll=4)
        def _edge(e):
            acc = srow[e, pl.ds(0, L)] * drow[e, pl.ds(0, L)]
            for k in range(1, D // L):
                acc = acc + srow[e, pl.ds(k * L, L)] * drow[e, pl.ds(k * L, L)]
            part[e, :] = acc

    def run(out_hbm):
        def wait_out(part, sem_o):
            pltpu.make_async_copy(
                part, out_hbm.at[pl.ds(hbase, CHUNK)], sem_o).wait()

        issue_idx(0, sidx_a, didx_a, sem_ia)
        issue_idx(1, sidx_b, didx_b, sem_ib)
        wait_idx(sidx_a, didx_a, sem_ia)
        issue_rows(sidx_a, didx_a, srow_a, drow_a, sem_sa, sem_da)

        def pair_body(i, carry):
            ga = 2 * i
            gb = ga + 1
            wait_idx(sidx_b, didx_b, sem_ib)
            issue_rows(sidx_b, didx_b, srow_b, drow_b, sem_sb, sem_db)

            wait_rows(sidx_a, didx_a, srow_a, drow_a, sem_sa, sem_da)

            @pl.when(ga + 2 < NCHUNK)
            def _idx_a():
                issue_idx(ga + 2, sidx_a, didx_a, sem_ia)

            @pl.when(i > 0)
            def _drain_a():
                wait_out(part_a, sem_oa)

            compute(srow_a, drow_a, part_a)
            pltpu.async_copy(
                part_a, out_hbm.at[pl.ds(hbase + ga * CHUNK, CHUNK)], sem_oa)

            wait_rows(sidx_b, didx_b, srow_b, drow_b, sem_sb, sem_db)

            @pl.when(gb + 2 < NCHUNK)
            def _idx_b():
                issue_idx(gb + 2, sidx_b, didx_b, sem_ib)

            @pl.when(i > 0)
            def _drain_b():
                wait_out(part_b, sem_ob)

            compute(srow_b, drow_b, part_b)
            pltpu.async_copy(
                part_b, out_hbm.at[pl.ds(hbase + gb * CHUNK, CHUNK)], sem_ob)

            @pl.when(ga + 2 < NCHUNK)
            def _rows_a():
                wait_idx(sidx_a, didx_a, sem_ia)
                issue_rows(sidx_a, didx_a, srow_a, drow_a, sem_sa, sem_da)

            return carry

        lax.fori_loop(0, NCHUNK // 2, pair_body, 0)
        wait_out(part_a, sem_oa)
        wait_out(part_b, sem_ob)

    @pl.when(cid == 0)
    def _core0():
        run(out0_hbm)

    @pl.when(cid == 1)
    def _core1():
        run(out1_hbm)


_BLK = 8000
_NBLK = E_HALF // _BLK


def _loss_body(p_pos_ref, p_neg_ref, out_ref):
    pid = pl.program_id(0)
    s0 = jnp.sum(p_pos_ref[...], axis=1, keepdims=True)
    s1 = jnp.sum(p_neg_ref[...], axis=1, keepdims=True)
    l0 = jnp.maximum(-s0, 0.0) + jnp.log1p(jnp.exp(-jnp.abs(s0)))
    l1 = jnp.maximum(s1, 0.0) + jnp.log1p(jnp.exp(-jnp.abs(s1)))
    bsum = jnp.sum(l0) + jnp.sum(l1)

    @pl.when(pid == 0)
    def _init():
        out_ref[0, 0] = 0.0

    out_ref[0, 0] += bsum

    @pl.when(pid == pl.num_programs(0) - 1)
    def _fini():
        out_ref[0, 0] = out_ref[0, 0] / E_TOT


_loss_tc = pl.pallas_call(
    _loss_body,
    grid=(_NBLK,),
    in_specs=[
        pl.BlockSpec((_BLK, L), lambda i: (i, 0)),
        pl.BlockSpec((_BLK, L), lambda i: (i, 0)),
    ],
    out_specs=pl.BlockSpec(
        (1, 1), lambda i: (0, 0), memory_space=pltpu.SMEM),
    out_shape=jax.ShapeDtypeStruct((1, 1), jnp.float32),
)


def kernel(block_outputs, pos_edge_index, neg_edge_index):
    src = jnp.concatenate(
        [pos_edge_index[0], neg_edge_index[0]]).astype(jnp.int32)
    dst = jnp.concatenate(
        [pos_edge_index[1], neg_edge_index[1]]).astype(jnp.int32)
    p_pos, p_neg = _edge_dot_sc(block_outputs, src, dst)
    loss = _loss_tc(p_pos, p_neg)
    return loss[0, 0]
